# baseline (speedup 1.0000x reference)
import functools

import jax
import jax.numpy as jnp
from jax import lax
from jax.experimental import pallas as pl
from jax.experimental.pallas import tpu as pltpu
from jax.experimental.pallas import tpu_sc as plsc

_LANES = 16


def _permute_cols(x, index):
    n_rows, n_cols = x.shape
    info = plsc.get_sparse_core_info()
    num_workers = info.num_cores * info.num_subcores
    rows_per_w = n_rows // num_workers
    chunk = 8
    while rows_per_w % (2 * chunk):
        chunk //= 2
    half = chunk // 2
    n_chunks = rows_per_w // chunk
    n_pairs = n_chunks // 2

    mesh = plsc.VectorSubcoreMesh(core_axis_name="c", subcore_axis_name="s")

    @functools.partial(
        pl.kernel,
        out_type=jax.ShapeDtypeStruct((n_rows, n_cols), jnp.float32),
        mesh=mesh,
        scratch_types=[
            pltpu.VMEM((n_cols,), jnp.int32),
            [pltpu.VMEM((chunk, n_cols), jnp.float32) for _ in range(2)],
            [pltpu.VMEM((half, n_cols), jnp.float32) for _ in range(2)],
            [pltpu.SemaphoreType.DMA for _ in range(2)],
            [pltpu.SemaphoreType.DMA for _ in range(2)],
        ],
        compiler_params=pltpu.CompilerParams(needs_layout_passes=False),
    )
    def run(x_hbm, idx_hbm, out_hbm, idx_v, in_v, out_v, sem_in, sem_out):
        wid = lax.axis_index("s") * info.num_cores + lax.axis_index("c")
        pltpu.sync_copy(idx_hbm, idx_v)
        base = wid * rows_per_w

        def src_at(ci):
            return x_hbm.at[pl.ds(base + ci * chunk, chunk)]

        def dst_at(ci, h):
            return out_hbm.at[pl.ds(base + ci * chunk + h * half, half)]

        for b in range(2):
            pltpu.async_copy(src_at(b), in_v[b], sem_in[b])

        def do_pair(pi, _):
            for b in range(2):
                ci = 2 * pi + b
                pltpu.make_async_copy(src_at(ci), in_v[b], sem_in[b]).wait()

                for h in range(2):
                    def drain(h=h, ci=ci):
                        pltpu.make_async_copy(
                            out_v[h], dst_at(ci - 1, h), sem_out[h]
                        ).wait()

                    if b > 0:
                        drain()
                    else:
                        pl.when(pi > 0)(drain)

                    @plsc.parallel_loop(0, n_cols, step=_LANES, unroll=8)
                    def gather_group(off):
                        off = pl.multiple_of(off, _LANES)
                        cols = idx_v[pl.ds(off, _LANES)]
                        for r in range(half):
                            row = jnp.full((_LANES,), h * half + r, jnp.int32)
                            vals = plsc.load_gather(in_v[b], [row, cols])
                            out_v[h][r, pl.ds(off, _LANES)] = vals

                    pltpu.async_copy(out_v[h], dst_at(ci, h), sem_out[h])

                @pl.when(pi < n_pairs - 1)
                def _():
                    pltpu.async_copy(src_at(ci + 2), in_v[b], sem_in[b])

            return 0

        lax.fori_loop(0, n_pairs, do_pair, 0)

        for h in range(2):
            pltpu.make_async_copy(
                out_v[h], dst_at(n_chunks - 1, h), sem_out[h]
            ).wait()

    return run(x, index)


def kernel(x, index):
    z = _permute_cols(x, index)
    log_det = jnp.zeros(x.shape[0], dtype=x.dtype)
    return (z, log_det)
